# baseline (speedup 1.0000x reference)
import functools

import jax
import jax.numpy as jnp
import numpy as np
from jax import lax
from jax.experimental import pallas as pl
from jax.experimental.pallas import tpu as pltpu
from jax.experimental.pallas import tpu_sc as plsc

N = 10000
NPAD = 10240
D = 128
B = 32
RATIO = 0.8

NC, NS = 2, 16
NW = NC * NS
CH = 80
ROWS_PER_S = NPAD // NS


def _mono_i32(x: float) -> int:
    i = int(np.float32(x).view(np.int32))
    return i if i >= 0 else i ^ 0x7FFFFFFF


_LO0 = _mono_i32(-1.0) - 1
_HI0 = _mono_i32(1.0)
_IMIN = -(2 ** 31)


def _sc_aggregate(h, src3, dst3, zeros_rows):
    n_chunks = src3.shape[1]
    n_a = n_chunks // 2
    n_b = n_chunks - n_a
    mesh = plsc.VectorSubcoreMesh(core_axis_name="c", subcore_axis_name="s")

    @functools.partial(
        pl.kernel,
        mesh=mesh,
        out_type=jax.ShapeDtypeStruct((NC, NPAD, D), jnp.float32),
        scratch_types=[
            pltpu.VMEM((n_b, CH), jnp.int32),
            pltpu.VMEM((n_b, CH), jnp.int32),
            pltpu.VMEM((2, CH, D), jnp.float32),
            pltpu.VMEM_SHARED((NPAD, D), jnp.float32),
            pltpu.SemaphoreType.DMA,
            pltpu.SemaphoreType.DMA,
        ],
    )
    def body(h_hbm, src_hbm, dst_hbm, z_hbm, out_hbm, src_v, dst_v, rows_v,
             aggr_sh, sem0, sem1):
        c = lax.axis_index("c")
        s = lax.axis_index("s")
        wid = s * NC + c
        pltpu.sync_copy(z_hbm, aggr_sh.at[pl.ds(s * ROWS_PER_S, ROWS_PER_S)])
        plsc.subcore_barrier()

        def gather(j, slot, sem):
            pltpu.async_copy(h_hbm.at[src_v.at[j]], rows_v.at[slot], sem)

        def drain(slot, sem):
            pltpu.make_async_copy(h_hbm.at[src_v.at[0]], rows_v.at[slot],
                                  sem).wait()

        def scatter(j, slot):
            pltpu.sync_copy(rows_v.at[slot], aggr_sh.at[dst_v.at[j]],
                            add=True)

        def phase(base, n):
            pltpu.sync_copy(src_hbm.at[wid, pl.ds(base, n)],
                            src_v.at[pl.ds(0, n)])
            pltpu.sync_copy(dst_hbm.at[wid, pl.ds(base, n)],
                            dst_v.at[pl.ds(0, n)])
            npairs = n // 2
            gather(0, 0, sem0)
            gather(1, 1, sem1)

            def step(g, carry):
                j = 2 * g
                drain(0, sem0)
                scatter(j, 0)
                gather(j + 2, 0, sem0)
                drain(1, sem1)
                scatter(j + 1, 1)
                gather(j + 3, 1, sem1)
                return carry

            lax.fori_loop(0, npairs - 1, step, 0)
            j = 2 * (npairs - 1)
            drain(0, sem0)
            scatter(j, 0)
            drain(1, sem1)
            scatter(j + 1, 1)
            if n % 2:
                gather(n - 1, 0, sem0)
                drain(0, sem0)
                scatter(n - 1, 0)

        phase(0, n_a)
        phase(n_a, n_b)
        plsc.subcore_barrier()
        pltpu.sync_copy(
            aggr_sh.at[pl.ds(s * ROWS_PER_S, ROWS_PER_S)],
            out_hbm.at[c, pl.ds(s * ROWS_PER_S, ROWS_PER_S)],
        )

    return body(h, src3, dst3, zeros_rows)


def _dotT(a, bt, hp):
    return jax.lax.dot_general(a, bt, (((1,), (1,)), ((), ())), precision=hp)


def _layer_tc_body(aggr_ref, h_ref, wr_ref, br_ref, wt_ref, p_ref, alivec_ref,
                   aliver_ref, batchc_ref, segt_ref,
                   hout_ref, alivecout_ref, xl_ref):
    hp = jax.lax.Precision.HIGHEST
    aggr = aggr_ref[0] + aggr_ref[1]
    h = h_ref[...]
    hn = aggr @ wr_ref[...] + br_ref[...] + h @ wt_ref[...]
    hn = jnp.maximum(hn, 0.0) * alivec_ref[...]
    p = p_ref[...]
    pn = jnp.sqrt(jnp.sum(p * p)) + 1e-16
    s_row = jnp.tanh(
        jax.lax.dot_general(p, hn, (((1,), (1,)), ((), ())), precision=hp)
        / pn)

    segt = segt_ref[...]
    aliver = aliver_ref[...]
    alive_b = aliver > 0.5
    cnt = _dotT(aliver, segt, hp)
    k_f = jnp.ceil(RATIO * cnt)

    bits = jax.lax.bitcast_convert_type(s_row, jnp.int32)
    key = jnp.where(bits >= 0, bits, bits ^ jnp.int32(0x7FFFFFFF))
    key = jnp.where(s_row == 0.0, jnp.int32(0), key)
    key_hi = jnp.where(alive_b, key, jnp.int32(_IMIN))

    def sel_nodes(vals):
        h16 = (vals >> 16).astype(jnp.float32)
        l16 = (vals & jnp.int32(0xFFFF)).astype(jnp.float32)
        hl = jnp.concatenate([h16, l16], axis=0)
        sel = jnp.dot(hl, segt, precision=hp)
        return ((sel[0:1].astype(jnp.int32) << 16)
                + sel[1:2].astype(jnp.int32))

    def search(keys, mask, kneed, lo0, hi0, iters):
        def step(_, carry):
            lo, hi = carry
            mid = hi - ((hi - lo) >> 1)
            mid_n = sel_nodes(mid)
            ind = jnp.where(mask & (keys >= mid_n), 1.0, 0.0)
            c = _dotT(ind, segt, hp)
            ok = c >= kneed
            return jnp.where(ok, mid, lo), jnp.where(ok, hi, mid - 1)
        lo, hi = lax.fori_loop(0, iters, step,
                               (jnp.full((1, B), lo0, jnp.int32),
                                jnp.full((1, B), hi0, jnp.int32)))
        return lo

    t_hi = search(key_hi, alive_b, k_f, _LO0, _HI0, 32)
    t_hi_n = sel_nodes(t_hi)
    gt = alive_b & (key_hi > t_hi_n)
    n_gt = _dotT(jnp.where(gt, 1.0, 0.0), segt, hp)
    need = k_f - n_gt
    tie = alive_b & (key_hi == t_hi_n)

    key_lo = (NPAD - 1) - jax.lax.broadcasted_iota(jnp.int32, (1, NPAD), 1)
    t_lo = search(key_lo, tie, need, 0, NPAD, 15)
    t_lo_n = sel_nodes(t_lo)

    kpos = (k_f > 0.5).astype(jnp.float32)
    kpos_n = jnp.dot(kpos, segt, precision=hp) > 0.5
    keep_row = kpos_n & (gt | (tie & (key_lo >= t_lo_n)))
    keepf_row = jnp.where(keep_row, 1.0, 0.0)
    gate_row = jnp.where(keep_row, s_row, 0.0)

    gate_col = jnp.transpose(gate_row)
    keep_col = jnp.transpose(keepf_row)
    h_out = hn * gate_col
    hout_ref[...] = h_out
    alivecout_ref[...] = keep_col

    sm = jnp.dot(segt, h_out, precision=hp)
    cnt2 = _dotT(keepf_row, segt, hp)
    mean = sm / jnp.maximum(jnp.transpose(cnt2), 1.0)

    nblk = NPAD // 128

    def mx_step(i, mx):
        blk = hout_ref[pl.ds(i * 128, 128), :]
        bb = batchc_ref[pl.ds(i * 128, 128), :]
        kc = alivecout_ref[pl.ds(i * 128, 128), :]
        rows = []
        for b in range(B):
            m = (bb == b) & (kc > 0.5)
            rows.append(jnp.max(jnp.where(m, blk, -1e30), axis=0,
                                keepdims=True))
        return jnp.maximum(mx, jnp.concatenate(rows, axis=0))

    mx = lax.fori_loop(0, nblk, mx_step, jnp.full((B, D), -1e30, jnp.float32))
    xl_ref[...] = jnp.concatenate([mx, mean], axis=1)


def _layer_tc(aggr, h, wr, br, wt, p, alivec, aliver, batchc, segt):
    return pl.pallas_call(
        _layer_tc_body,
        out_shape=[
            jax.ShapeDtypeStruct((NPAD, D), jnp.float32),
            jax.ShapeDtypeStruct((NPAD, 1), jnp.float32),
            jax.ShapeDtypeStruct((B, 2 * D), jnp.float32),
        ],
    )(aggr, h, wr, br, wt, p, alivec, aliver, batchc, segt)


def _head_tc_body(g_ref, w1_ref, b1_ref, w2_ref, b2_ref, w3_ref, b3_ref,
                  out_ref):
    g = g_ref[...]
    g = jnp.maximum(g @ w1_ref[...] + b1_ref[...], 0.0)
    g = jnp.maximum(g @ w2_ref[...] + b2_ref[...], 0.0)
    z = g @ w3_ref[...] + b3_ref[...]
    col = jax.lax.broadcasted_iota(jnp.int32, (B, 128), 1)
    z = jnp.where(col < 10, z, -1e30)
    m = jnp.max(z, axis=1, keepdims=True)
    lse = m + jnp.log(jnp.sum(jnp.exp(z - m), axis=1, keepdims=True))
    out_ref[...] = z - lse


def _head_tc(g, w1, b1, w2, b2, w3, b3):
    return pl.pallas_call(
        _head_tc_body,
        out_shape=jax.ShapeDtypeStruct((B, 128), jnp.float32),
    )(g, w1, b1, w2, b2, w3, b3)


def kernel(x, edge_index, batch, W_rel1, b_rel1, W_root1, p1, W_rel2, b_rel2,
           W_root2, p2, W_rel3, b_rel3, W_root3, p3, W_lin1, b_lin1, W_lin2,
           b_lin2, W_lin3, b_lin3):
    E = edge_index.shape[1]
    n_chunks = -(-(E // NW) // (8 * CH)) * 8
    epad = NW * n_chunks * CH - E
    src3 = jnp.pad(edge_index[0], (0, epad),
                   constant_values=NPAD - 1).reshape(NW, n_chunks, CH)
    dst3 = jnp.pad(edge_index[1], (0, epad),
                   constant_values=NPAD - 1).reshape(NW, n_chunks, CH)

    xp = jnp.zeros((NPAD, D), jnp.float32).at[:N].set(x)
    node_id = jnp.arange(NPAD, dtype=jnp.int32)
    alive_flat = (node_id < N).astype(jnp.float32)
    batch_p = jnp.where(node_id < N, jnp.pad(batch, (0, NPAD - N)), B)
    segt = (batch_p[None, :] == jnp.arange(B)[:, None]).astype(jnp.float32)
    batchc = batch_p[:, None].astype(jnp.int32)
    zeros_rows = jnp.zeros((ROWS_PER_S, D), jnp.float32)

    layers = (
        (W_rel1, b_rel1, W_root1, p1),
        (W_rel2, b_rel2, W_root2, p2),
        (W_rel3, b_rel3, W_root3, p3),
    )
    h = xp
    alivec = alive_flat[:, None]
    aliver = alive_flat[None, :]
    xs = []
    for (wr, br, wt, p) in layers:
        parts = _sc_aggregate(h, src3, dst3, zeros_rows)
        h, alivec, xl = _layer_tc(parts, h, wr, br[None, :], wt, p[None, :],
                                  alivec, aliver, batchc, segt)
        aliver = alivec.reshape(1, NPAD)
        xs.append(xl)
    g = xs[0] + xs[1] + xs[2]

    w2p = jnp.zeros((128, 128), jnp.float32).at[:, :64].set(W_lin2)
    b2p = jnp.zeros((128,), jnp.float32).at[:64].set(b_lin2)
    w3p = jnp.zeros((128, 128), jnp.float32).at[:64, :10].set(W_lin3)
    b3p = jnp.zeros((128,), jnp.float32).at[:10].set(b_lin3)
    out = _head_tc(g, W_lin1, b_lin1[None, :], w2p, b2p[None, :], w3p,
                   b3p[None, :])
    return out[:, :10]


# speedup vs baseline: 2.1201x; 2.1201x over previous
import functools

import jax
import jax.numpy as jnp
import numpy as np
from jax import lax
from jax.experimental import pallas as pl
from jax.experimental.pallas import tpu as pltpu
from jax.experimental.pallas import tpu_sc as plsc

N = 10000
NPAD = 10240
D = 128
B = 32
RATIO = 0.8

NC, NS = 2, 16
NW = NC * NS
CH = 80
ROWS_PER_S = NPAD // NS


def _mono_i32(x: float) -> int:
    i = int(np.float32(x).view(np.int32))
    return i if i >= 0 else i ^ 0x7FFFFFFF


_LO0 = _mono_i32(-1.0) - 1
_HI0 = _mono_i32(1.0)
_IMIN = -(2 ** 31)


def _sc_aggregate(h, src3, dst3, zeros_rows):
    n_chunks = src3.shape[1]
    n_a = n_chunks // 2
    n_b = n_chunks - n_a
    mesh = plsc.VectorSubcoreMesh(core_axis_name="c", subcore_axis_name="s")

    @functools.partial(
        pl.kernel,
        mesh=mesh,
        out_type=jax.ShapeDtypeStruct((NC, NPAD, D), jnp.float32),
        scratch_types=[
            pltpu.VMEM((n_b, CH), jnp.int32),
            pltpu.VMEM((n_b, CH), jnp.int32),
            pltpu.VMEM((2, CH, D), jnp.float32),
            pltpu.VMEM_SHARED((NPAD, D), jnp.float32),
            pltpu.SemaphoreType.DMA,
            pltpu.SemaphoreType.DMA,
        ],
    )
    def body(h_hbm, src_hbm, dst_hbm, z_hbm, out_hbm, src_v, dst_v, rows_v,
             aggr_sh, sem0, sem1):
        c = lax.axis_index("c")
        s = lax.axis_index("s")
        wid = s * NC + c
        pltpu.sync_copy(z_hbm, aggr_sh.at[pl.ds(s * ROWS_PER_S, ROWS_PER_S)])
        plsc.subcore_barrier()

        def gather(j, slot, sem):
            pltpu.async_copy(h_hbm.at[src_v.at[j]], rows_v.at[slot], sem)

        def drain(slot, sem):
            pltpu.make_async_copy(h_hbm.at[src_v.at[0]], rows_v.at[slot],
                                  sem).wait()

        def scatter(j, slot):
            pltpu.sync_copy(rows_v.at[slot], aggr_sh.at[dst_v.at[j]],
                            add=True)

        def phase(base, n):
            pltpu.sync_copy(src_hbm.at[wid, pl.ds(base, n)],
                            src_v.at[pl.ds(0, n)])
            pltpu.sync_copy(dst_hbm.at[wid, pl.ds(base, n)],
                            dst_v.at[pl.ds(0, n)])
            npairs = n // 2
            gather(0, 0, sem0)
            gather(1, 1, sem1)

            def step(g, carry):
                j = 2 * g
                drain(0, sem0)
                scatter(j, 0)
                gather(j + 2, 0, sem0)
                drain(1, sem1)
                scatter(j + 1, 1)
                gather(j + 3, 1, sem1)
                return carry

            lax.fori_loop(0, npairs - 1, step, 0)
            j = 2 * (npairs - 1)
            drain(0, sem0)
            scatter(j, 0)
            drain(1, sem1)
            scatter(j + 1, 1)
            if n % 2:
                gather(n - 1, 0, sem0)
                drain(0, sem0)
                scatter(n - 1, 0)

        phase(0, n_a)
        phase(n_a, n_b)
        plsc.subcore_barrier()
        pltpu.sync_copy(
            aggr_sh.at[pl.ds(s * ROWS_PER_S, ROWS_PER_S)],
            out_hbm.at[c, pl.ds(s * ROWS_PER_S, ROWS_PER_S)],
        )

    return body(h, src3, dst3, zeros_rows)


def _dotT(a, bt, hp):
    return jax.lax.dot_general(a, bt, (((1,), (1,)), ((), ())), precision=hp)


def _layer_tc_body(aggr_ref, h_ref, wr_ref, br_ref, wt_ref, p_ref, alivec_ref,
                   aliver_ref, batchc_ref, segt_ref,
                   hout_ref, alivecout_ref, xl_ref):
    hp = jax.lax.Precision.HIGHEST
    aggr = aggr_ref[0] + aggr_ref[1]
    h = h_ref[...]
    hn = aggr @ wr_ref[...] + br_ref[...] + h @ wt_ref[...]
    hn = jnp.maximum(hn, 0.0) * alivec_ref[...]
    p = p_ref[...]
    pn = jnp.sqrt(jnp.sum(p * p)) + 1e-16
    s_row = jnp.tanh(
        jax.lax.dot_general(p, hn, (((1,), (1,)), ((), ())), precision=hp)
        / pn)

    segt = segt_ref[...]
    aliver = aliver_ref[...]
    alive_b = aliver > 0.5
    cnt = _dotT(aliver, segt, hp)
    k_f = jnp.ceil(RATIO * cnt)

    bits = jax.lax.bitcast_convert_type(s_row, jnp.int32)
    key = jnp.where(bits >= 0, bits, bits ^ jnp.int32(0x7FFFFFFF))
    key = jnp.where(s_row == 0.0, jnp.int32(0), key)
    key_hi = jnp.where(alive_b, key, jnp.int32(_IMIN))

    def sel_nodes(vals):
        h16 = (vals >> 16).astype(jnp.float32)
        l16 = (vals & jnp.int32(0xFFFF)).astype(jnp.float32)
        hl = jnp.concatenate([h16, l16], axis=0)
        sel = jnp.dot(hl, segt, precision=hp)
        return ((sel[0:1].astype(jnp.int32) << 16)
                + sel[1:2].astype(jnp.int32))

    def search(keys, mask, kneed, lo0, hi0, iters):
        def step(_, carry):
            lo, hi = carry
            mid = hi - ((hi - lo) >> 1)
            mid_n = sel_nodes(mid)
            ind = jnp.where(mask & (keys >= mid_n), 1.0, 0.0)
            c = _dotT(ind, segt, hp)
            ok = c >= kneed
            return jnp.where(ok, mid, lo), jnp.where(ok, hi, mid - 1)
        lo, hi = lax.fori_loop(0, iters, step,
                               (jnp.full((1, B), lo0, jnp.int32),
                                jnp.full((1, B), hi0, jnp.int32)))
        return lo

    t_hi = search(key_hi, alive_b, k_f, _LO0, _HI0, 32)
    t_hi_n = sel_nodes(t_hi)
    gt = alive_b & (key_hi > t_hi_n)
    n_gt = _dotT(jnp.where(gt, 1.0, 0.0), segt, hp)
    need = k_f - n_gt
    tie = alive_b & (key_hi == t_hi_n)

    key_lo = (NPAD - 1) - jax.lax.broadcasted_iota(jnp.int32, (1, NPAD), 1)
    t_lo = search(key_lo, tie, need, 0, NPAD, 15)
    t_lo_n = sel_nodes(t_lo)

    kpos = (k_f > 0.5).astype(jnp.float32)
    kpos_n = jnp.dot(kpos, segt, precision=hp) > 0.5
    keep_row = kpos_n & (gt | (tie & (key_lo >= t_lo_n)))
    keepf_row = jnp.where(keep_row, 1.0, 0.0)
    gate_row = jnp.where(keep_row, s_row, 0.0)

    gate_col = jnp.transpose(gate_row)
    keep_col = jnp.transpose(keepf_row)
    h_out = hn * gate_col
    hout_ref[...] = h_out
    alivecout_ref[...] = keep_col

    sm = jnp.dot(segt, h_out, precision=hp)
    cnt2 = _dotT(keepf_row, segt, hp)
    mean = sm / jnp.maximum(jnp.transpose(cnt2), 1.0)

    nblk = NPAD // 128

    def mx_step(i, mx):
        blk = hout_ref[pl.ds(i * 128, 128), :]
        bb = batchc_ref[pl.ds(i * 128, 128), :]
        kc = alivecout_ref[pl.ds(i * 128, 128), :]
        rows = []
        for b in range(B):
            m = (bb == b) & (kc > 0.5)
            rows.append(jnp.max(jnp.where(m, blk, -1e30), axis=0,
                                keepdims=True))
        return jnp.maximum(mx, jnp.concatenate(rows, axis=0))

    mx = lax.fori_loop(0, nblk, mx_step, jnp.full((B, D), -1e30, jnp.float32))
    xl_ref[...] = jnp.concatenate([mx, mean], axis=1)


def _layer_tc(aggr, h, wr, br, wt, p, alivec, aliver, batchc, segt):
    return pl.pallas_call(
        _layer_tc_body,
        out_shape=[
            jax.ShapeDtypeStruct((NPAD, D), jnp.float32),
            jax.ShapeDtypeStruct((NPAD, 1), jnp.float32),
            jax.ShapeDtypeStruct((B, 2 * D), jnp.float32),
        ],
    )(aggr, h, wr, br, wt, p, alivec, aliver, batchc, segt)


def _head_tc_body(g_ref, w1_ref, b1_ref, w2_ref, b2_ref, w3_ref, b3_ref,
                  out_ref):
    g = g_ref[...]
    g = jnp.maximum(g @ w1_ref[...] + b1_ref[...], 0.0)
    g = jnp.maximum(g @ w2_ref[...] + b2_ref[...], 0.0)
    z = g @ w3_ref[...] + b3_ref[...]
    col = jax.lax.broadcasted_iota(jnp.int32, (B, 128), 1)
    z = jnp.where(col < 10, z, -1e30)
    m = jnp.max(z, axis=1, keepdims=True)
    lse = m + jnp.log(jnp.sum(jnp.exp(z - m), axis=1, keepdims=True))
    out_ref[...] = z - lse


def _head_tc(g, w1, b1, w2, b2, w3, b3):
    return pl.pallas_call(
        _head_tc_body,
        out_shape=jax.ShapeDtypeStruct((B, 128), jnp.float32),
    )(g, w1, b1, w2, b2, w3, b3)


def kernel(x, edge_index, batch, W_rel1, b_rel1, W_root1, p1, W_rel2, b_rel2,
           W_root2, p2, W_rel3, b_rel3, W_root3, p3, W_lin1, b_lin1, W_lin2,
           b_lin2, W_lin3, b_lin3):
    E = edge_index.shape[1]
    per_w = E // NW
    n_chunks = -(-per_w // (8 * CH)) * 8
    wpad = n_chunks * CH - per_w
    dummy = (N + jnp.arange(NW * wpad, dtype=jnp.int32) % (NPAD - N)
             ).reshape(NW, wpad)

    def shard(e):
        return jnp.concatenate([e.reshape(NW, per_w), dummy],
                               axis=1).reshape(NW, n_chunks, CH)

    src3 = shard(edge_index[0])
    dst3 = shard(edge_index[1])

    xp = jnp.zeros((NPAD, D), jnp.float32).at[:N].set(x)
    node_id = jnp.arange(NPAD, dtype=jnp.int32)
    alive_flat = (node_id < N).astype(jnp.float32)
    batch_p = jnp.where(node_id < N, jnp.pad(batch, (0, NPAD - N)), B)
    segt = (batch_p[None, :] == jnp.arange(B)[:, None]).astype(jnp.float32)
    batchc = batch_p[:, None].astype(jnp.int32)
    zeros_rows = jnp.zeros((ROWS_PER_S, D), jnp.float32)

    layers = (
        (W_rel1, b_rel1, W_root1, p1),
        (W_rel2, b_rel2, W_root2, p2),
        (W_rel3, b_rel3, W_root3, p3),
    )
    h = xp
    alivec = alive_flat[:, None]
    aliver = alive_flat[None, :]
    xs = []
    for (wr, br, wt, p) in layers:
        parts = _sc_aggregate(h, src3, dst3, zeros_rows)
        h, alivec, xl = _layer_tc(parts, h, wr, br[None, :], wt, p[None, :],
                                  alivec, aliver, batchc, segt)
        aliver = alivec.reshape(1, NPAD)
        xs.append(xl)
    g = xs[0] + xs[1] + xs[2]

    w2p = jnp.zeros((128, 128), jnp.float32).at[:, :64].set(W_lin2)
    b2p = jnp.zeros((128,), jnp.float32).at[:64].set(b_lin2)
    w3p = jnp.zeros((128, 128), jnp.float32).at[:64, :10].set(W_lin3)
    b3p = jnp.zeros((128,), jnp.float32).at[:10].set(b_lin3)
    out = _head_tc(g, W_lin1, b_lin1[None, :], w2p, b2p[None, :], w3p,
                   b3p[None, :])
    return out[:, :10]
